# baseline (speedup 1.0000x reference)
import functools

import jax
import jax.numpy as jnp
from jax import lax
from jax.experimental import pallas as pl
from jax.experimental.pallas import tpu as pltpu
from jax.experimental.pallas import tpu_sc as plsc

_EPS = 1e-12
_K = 8192
_CD = 256
_D = 768


def _out_proj_body(ze_ref, zq_ref, vout_ref, gout_ref, bout_ref, out_ref,
                   sq_ref):
    v = vout_ref[...]
    norm = jnp.sqrt(jnp.sum(v * v, axis=0, keepdims=True))
    w = gout_ref[...] * v / jnp.maximum(norm, _EPS)
    ze = ze_ref[0]
    zq = zq_ref[0]
    qst = ze + (zq - ze)
    out_ref[...] = (jnp.dot(qst, w, preferred_element_type=jnp.float32)
                    + bout_ref[...])[None]
    d = ze - zq
    sq_ref[...] = jnp.sum(d * d, axis=0, keepdims=True)[None]


def _out_proj_pass(ze3, zq3, v_out, g_out, b_out):
    B, T, _ = ze3.shape
    return pl.pallas_call(
        _out_proj_body,
        grid=(B,),
        in_specs=[
            pl.BlockSpec((1, T, _CD), lambda b: (b, 0, 0)),
            pl.BlockSpec((1, T, _CD), lambda b: (b, 0, 0)),
            pl.BlockSpec((_CD, _D), lambda b: (0, 0)),
            pl.BlockSpec((1, _D), lambda b: (0, 0)),
            pl.BlockSpec((1, _D), lambda b: (0, 0)),
        ],
        out_specs=[
            pl.BlockSpec((1, T, _D), lambda b: (b, 0, 0)),
            pl.BlockSpec((1, 1, _CD), lambda b: (b, 0, 0)),
        ],
        out_shape=[
            jax.ShapeDtypeStruct((B, T, _D), jnp.float32),
            jax.ShapeDtypeStruct((B, 1, _CD), jnp.float32),
        ],
    )(ze3, zq3, v_out, g_out, b_out)


def _sc_gather(codebook, idx):
    info = plsc.get_sparse_core_info()
    nw = info.num_cores * info.num_subcores
    n = idx.shape[0]
    bpw = n // nw
    ch = 96
    nch = bpw // ch
    mesh = plsc.VectorSubcoreMesh(core_axis_name="c", subcore_axis_name="s")

    @functools.partial(
        pl.kernel, mesh=mesh,
        out_type=jax.ShapeDtypeStruct((n, _CD), jnp.float32),
        scratch_types=[
            pltpu.VMEM((bpw,), jnp.int32),
            pltpu.VMEM((bpw, _CD), jnp.float32),
            pltpu.SemaphoreType.DMA,
        ],
    )
    def k(idx_hbm, table_hbm, out_hbm, idx_v, rows_v, sem):
        wid = lax.axis_index("s") * info.num_cores + lax.axis_index("c")
        base = wid * bpw
        pltpu.sync_copy(idx_hbm.at[pl.ds(base, bpw)], idx_v)
        for j in range(nch):
            pltpu.async_copy(
                table_hbm.at[idx_v.at[pl.ds(j * ch, ch)]],
                rows_v.at[pl.ds(j * ch, ch)], sem).wait()
        pltpu.sync_copy(rows_v, out_hbm.at[pl.ds(base, bpw)])

    return k(idx, codebook)


def kernel(z, v_in, g_in, b_in, v_out, g_out, b_out, codebook):
    B, T, D = z.shape

    norm_in = jnp.sqrt(jnp.sum(v_in * v_in, axis=0, keepdims=True))
    w_in = g_in[None, :] * v_in / jnp.maximum(norm_in, _EPS)
    z_e = jnp.einsum('btd,dc->btc', z, w_in) + b_in
    encodings = z_e.reshape(B * T, _CD)
    den_e = jnp.maximum(_EPS, jnp.linalg.norm(encodings, ord=2, axis=1,
                                              keepdims=True))
    enc_n = encodings / den_e
    den_c = jnp.maximum(_EPS, jnp.linalg.norm(codebook, ord=2, axis=1,
                                              keepdims=True))
    cb_n = codebook / den_c
    dist = (jnp.square(enc_n).sum(1, keepdims=True)
            - 2.0 * enc_n @ cb_n.T
            + jnp.square(cb_n).sum(1, keepdims=True).T)
    indices = jnp.argmax(-dist, axis=1).reshape(B, T)

    zq3 = jnp.take(codebook, indices, axis=0)
    out, sq = _out_proj_pass(z_e, zq3, v_out, g_out[None, :], b_out[None, :])

    loss = jnp.sum(sq, axis=(1, 2)) / jnp.float32(T * _CD)
    return (out, loss, loss, indices, z_e)


# speedup vs baseline: 1.0009x; 1.0009x over previous
import jax
import jax.numpy as jnp
from jax.experimental import pallas as pl

_EPS = 1e-12
_CD = 256
_D = 768


def _out_proj_body(ze_ref, zq_ref, vout_ref, gout_ref, bout_ref, out_ref,
                   sq_ref):
    v = vout_ref[...]
    norm = jnp.sqrt(jnp.sum(v * v, axis=0, keepdims=True))
    w = gout_ref[...] * v / jnp.maximum(norm, _EPS)
    ze = ze_ref[0]
    zq = zq_ref[0]
    qst = ze + (zq - ze)
    out_ref[...] = (jnp.dot(qst, w, preferred_element_type=jnp.float32)
                    + bout_ref[...])[None]
    d = ze - zq
    sq_ref[...] = jnp.sum(d * d, axis=0, keepdims=True)[None]


def _out_proj_pass(ze3, zq3, v_out, g_out, b_out):
    B, T, _ = ze3.shape
    return pl.pallas_call(
        _out_proj_body,
        grid=(B,),
        in_specs=[
            pl.BlockSpec((1, T, _CD), lambda b: (b, 0, 0)),
            pl.BlockSpec((1, T, _CD), lambda b: (b, 0, 0)),
            pl.BlockSpec((_CD, _D), lambda b: (0, 0)),
            pl.BlockSpec((1, _D), lambda b: (0, 0)),
            pl.BlockSpec((1, _D), lambda b: (0, 0)),
        ],
        out_specs=[
            pl.BlockSpec((1, T, _D), lambda b: (b, 0, 0)),
            pl.BlockSpec((1, 1, _CD), lambda b: (b, 0, 0)),
        ],
        out_shape=[
            jax.ShapeDtypeStruct((B, T, _D), jnp.float32),
            jax.ShapeDtypeStruct((B, 1, _CD), jnp.float32),
        ],
    )(ze3, zq3, v_out, g_out, b_out)


def kernel(z, v_in, g_in, b_in, v_out, g_out, b_out, codebook):
    B, T, D = z.shape

    norm_in = jnp.sqrt(jnp.sum(v_in * v_in, axis=0, keepdims=True))
    w_in = g_in[None, :] * v_in / jnp.maximum(norm_in, _EPS)
    z_e = jnp.einsum('btd,dc->btc', z, w_in) + b_in
    encodings = z_e.reshape(B * T, _CD)
    den_e = jnp.maximum(_EPS, jnp.linalg.norm(encodings, ord=2, axis=1,
                                              keepdims=True))
    enc_n = encodings / den_e
    den_c = jnp.maximum(_EPS, jnp.linalg.norm(codebook, ord=2, axis=1,
                                              keepdims=True))
    cb_n = codebook / den_c
    dist = (jnp.square(enc_n).sum(1, keepdims=True)
            - 2.0 * enc_n @ cb_n.T
            + jnp.square(cb_n).sum(1, keepdims=True).T)
    indices = jnp.argmax(-dist, axis=1).reshape(B, T)

    zq3 = jnp.take(codebook, indices, axis=0)
    out, sq = _out_proj_pass(z_e, zq3, v_out, g_out[None, :], b_out[None, :])

    loss = jnp.sum(sq, axis=(1, 2)) / jnp.float32(T * _CD)
    return (out, loss, loss, indices, z_e)
